# Initial kernel scaffold; baseline (speedup 1.0000x reference)
import jax
import jax.numpy as jnp
from jax.experimental import pallas as pl


def kernel(x, W_org, b_org, W_gate, lora_update, lora_degen, bias_w, debias_w):
    raise NotImplementedError("write your pallas kernel here")


# speedup vs baseline: 4.1350x; 4.1350x over previous
import jax
import jax.numpy as jnp
from jax.experimental import pallas as pl

C = 16
R = 2
GR = 8

_PREC = jax.lax.Precision.DEFAULT


def _fused(x_ref, w_ref, b_ref, wg_ref, u_ref, db32_ref, dg_ref, bias_ref,
           o_ref):
    m = x_ref.shape[0]
    x1 = jax.lax.dot_general(
        x_ref[...], w_ref[...], (((1,), (1,)), ((), ())),
        precision=_PREC, preferred_element_type=jnp.float32)
    x1 = x1 + b_ref[...]

    proj = jax.lax.dot_general(
        x1, wg_ref[...], (((1,), (0,)), ((), ())),
        precision=_PREC, preferred_element_type=jnp.float32)
    proj2 = proj * proj
    srow = jax.lax.broadcasted_iota(jnp.int32, (C * GR, C), 0) // GR
    scol = jax.lax.broadcasted_iota(jnp.int32, (C * GR, C), 1)
    sel = (srow == scol).astype(jnp.float32)
    score = jax.lax.dot_general(
        proj2, sel, (((1,), (0,)), ((), ())),
        precision=jax.lax.Precision.HIGHEST,
        preferred_element_type=jnp.float32)

    idx = jnp.argmax(score, axis=-1)
    smax = jnp.max(score, axis=-1, keepdims=True)
    sg = jax.nn.sigmoid(smax)

    lane16 = jax.lax.broadcasted_iota(jnp.int32, (m, C), 1)
    g16 = (lane16 == idx[:, None]).astype(jnp.float32)
    lane32 = jax.lax.broadcasted_iota(jnp.int32, (m, C * R), 1)
    g32 = (lane32 // R == idx[:, None]).astype(jnp.float32)

    P = jax.lax.dot_general(
        x1, u_ref[...], (((1,), (0,)), ((), ())),
        precision=_PREC, preferred_element_type=jnp.float32)
    d2u = jnp.sum(db32_ref[...] * u_ref[...], axis=0, keepdims=True)
    Q = g32 * (P - d2u)
    mod = jax.lax.dot_general(
        Q, dg_ref[...], (((1,), (0,)), ((), ())),
        precision=jax.lax.Precision.HIGHEST,
        preferred_element_type=jnp.float32)
    bias_sel = jax.lax.dot_general(
        g16, bias_ref[...], (((1,), (0,)), ((), ())),
        precision=jax.lax.Precision.HIGHEST,
        preferred_element_type=jnp.float32)

    o_ref[...] = (1.0 - sg) * x1 + sg * (bias_sel + mod)


def kernel(x, W_org, b_org, W_gate, lora_update, lora_degen, bias_w,
           debias_w):
    B, T, D = x.shape
    BT = B * T
    M = 512
    assert BT % M == 0

    xf = x.reshape(BT, D)
    b2 = b_org.reshape(1, D)
    Wg = W_gate.transpose(1, 0, 2).reshape(D, C * GR)
    U = lora_update.transpose(1, 0, 2).reshape(D, C * R)
    Db32 = jnp.repeat(debias_w.T, R, axis=1)
    Dg = lora_degen.transpose(0, 2, 1).reshape(C * R, D)

    out = pl.pallas_call(
        _fused,
        grid=(BT // M,),
        in_specs=[
            pl.BlockSpec((M, D), lambda i: (i, 0)),
            pl.BlockSpec((D, D), lambda i: (0, 0)),
            pl.BlockSpec((1, D), lambda i: (0, 0)),
            pl.BlockSpec((D, C * GR), lambda i: (0, 0)),
            pl.BlockSpec((D, C * R), lambda i: (0, 0)),
            pl.BlockSpec((D, C * R), lambda i: (0, 0)),
            pl.BlockSpec((C * R, D), lambda i: (0, 0)),
            pl.BlockSpec((C, D), lambda i: (0, 0)),
        ],
        out_specs=pl.BlockSpec((M, D), lambda i: (i, 0)),
        out_shape=jax.ShapeDtypeStruct((BT, D), jnp.float32),
    )(xf, W_org, b2, Wg, U, Db32, Dg, bias_w)
    return out.reshape(B, T, D)


# speedup vs baseline: 6.9681x; 1.6852x over previous
import jax
import jax.numpy as jnp
from jax.experimental import pallas as pl

C = 16
R = 2
GR = 8

_PREC = jax.lax.Precision.DEFAULT


def _fused(x_ref, w_ref, b_ref, wgu_ref, u_ref, db32_ref, wsel_ref, o_ref):
    m = x_ref.shape[0]
    x1 = jax.lax.dot_general(
        x_ref[...], w_ref[...], (((1,), (1,)), ((), ())),
        precision=_PREC, preferred_element_type=jnp.float32)
    x1 = x1 + b_ref[...]

    y = jax.lax.dot_general(
        x1, wgu_ref[...], (((1,), (0,)), ((), ())),
        precision=_PREC, preferred_element_type=jnp.float32)
    proj = y[:, :C * GR]
    P = y[:, C * GR:]
    proj2 = proj * proj
    srow = jax.lax.broadcasted_iota(jnp.int32, (C * GR, C), 0) // GR
    scol = jax.lax.broadcasted_iota(jnp.int32, (C * GR, C), 1)
    sel = (srow == scol).astype(jnp.float32)
    score = jax.lax.dot_general(
        proj2, sel, (((1,), (0,)), ((), ())),
        precision=jax.lax.Precision.HIGHEST,
        preferred_element_type=jnp.float32)

    idx = jnp.argmax(score, axis=-1)
    smax = jnp.max(score, axis=-1, keepdims=True)
    sg = jax.nn.sigmoid(smax)

    lane16 = jax.lax.broadcasted_iota(jnp.int32, (m, C), 1)
    g16 = (lane16 == idx[:, None]).astype(jnp.float32)
    lane32 = jax.lax.broadcasted_iota(jnp.int32, (m, C * R), 1)
    g32 = (lane32 // R == idx[:, None]).astype(jnp.float32)

    d2u = jnp.sum(db32_ref[...] * u_ref[...], axis=0, keepdims=True)
    Q = g32 * (P - d2u)
    selcat = jnp.concatenate([g16, Q], axis=1)
    biasmod = jax.lax.dot_general(
        selcat, wsel_ref[...], (((1,), (0,)), ((), ())),
        precision=_PREC, preferred_element_type=jnp.float32)

    o_ref[...] = x1 + sg * (biasmod - x1)


def kernel(x, W_org, b_org, W_gate, lora_update, lora_degen, bias_w,
           debias_w):
    B, T, D = x.shape
    BT = B * T
    M = 512
    assert BT % M == 0

    xf = x.reshape(BT, D)
    b2 = b_org.reshape(1, D)
    Wg = W_gate.transpose(1, 0, 2).reshape(D, C * GR)
    U = lora_update.transpose(1, 0, 2).reshape(D, C * R)
    WgU = jnp.concatenate([Wg, U], axis=1)
    Db32 = jnp.repeat(debias_w.T, R, axis=1)
    Dg = lora_degen.transpose(0, 2, 1).reshape(C * R, D)
    Wsel = jnp.concatenate([bias_w, Dg], axis=0)

    out = pl.pallas_call(
        _fused,
        grid=(BT // M,),
        in_specs=[
            pl.BlockSpec((M, D), lambda i: (i, 0)),
            pl.BlockSpec((D, D), lambda i: (0, 0)),
            pl.BlockSpec((1, D), lambda i: (0, 0)),
            pl.BlockSpec((D, C * (GR + R)), lambda i: (0, 0)),
            pl.BlockSpec((D, C * R), lambda i: (0, 0)),
            pl.BlockSpec((D, C * R), lambda i: (0, 0)),
            pl.BlockSpec((C * (1 + R), D), lambda i: (0, 0)),
        ],
        out_specs=pl.BlockSpec((M, D), lambda i: (i, 0)),
        out_shape=jax.ShapeDtypeStruct((BT, D), jnp.float32),
    )(xf, W_org, b2, WgU, U, Db32, Wsel)
    return out.reshape(B, T, D)
